# Initial kernel scaffold; baseline (speedup 1.0000x reference)
import jax
import jax.numpy as jnp
from jax.experimental import pallas as pl


def kernel(x, edge_index, edge_attr, batch, W1e, b1e, root1, bias1, W2e, b2e, root2, bias2, Wp1, bp1, Wp2, bp2):
    raise NotImplementedError("write your pallas kernel here")


# speedup vs baseline: 1.9214x; 1.9214x over previous
import functools

import jax
import jax.numpy as jnp
from jax import lax
from jax.experimental import pallas as pl
from jax.experimental.pallas import tpu as pltpu
from jax.experimental.pallas import tpu_sc as plsc

N = 10000
E = 160000
F = 32
ED = 16
G = 64

NC = 2
NS = 16
NW = NC * NS
EPW = E // NW
CHUNK = 125
NCH = EPW // CHUNK
CPH = 20
NGRP = NCH // CPH
NPT = N // NS

TB = 2000

@functools.cache
def _sc_mesh():
    return plsc.VectorSubcoreMesh(core_axis_name="c", subcore_axis_name="s",
                                  num_cores=NC, num_subcores=NS)


def _gather_body(table_hbm, idx_hbm, out_hbm, idx_v, rows_v, sem):
    c = lax.axis_index("c")
    s = lax.axis_index("s")
    wid = s * NC + c
    pltpu.sync_copy(idx_hbm.at[wid], idx_v)
    for g in range(NGRP):
        cps = [
            pltpu.async_copy(table_hbm.at[idx_v.at[g * CPH + j]], rows_v.at[j], sem)
            for j in range(CPH)
        ]
        for cp in cps:
            cp.wait()
        pltpu.sync_copy(rows_v, out_hbm.at[wid].at[pl.ds(g * CPH, CPH)])


def _sc_gather(table, idx3):
    return pl.kernel(
        _gather_body,
        out_type=jax.ShapeDtypeStruct((NW, NCH, CHUNK, F), jnp.float32),
        mesh=_sc_mesh(),
        scratch_types=[
            pltpu.VMEM((NCH, CHUNK), jnp.int32),
            pltpu.VMEM((CPH, CHUNK, F), jnp.float32),
            pltpu.SemaphoreType.DMA,
        ],
        compiler_params=pltpu.CompilerParams(use_tc_tiling_on_sc=False),
    )(table, idx3)


def _scatter_body(msg_hbm, idx_hbm, z_hbm, out_hbm, idx_v, rows_v, acc_sh, sem):
    c = lax.axis_index("c")
    s = lax.axis_index("s")
    wid = s * NC + c
    pltpu.sync_copy(z_hbm.at[pl.ds(s * NPT, NPT)], acc_sh.at[pl.ds(s * NPT, NPT)])
    pltpu.sync_copy(idx_hbm.at[wid], idx_v)
    plsc.subcore_barrier()
    for g in range(NGRP):
        pltpu.sync_copy(msg_hbm.at[wid].at[pl.ds(g * CPH, CPH)], rows_v)
        for j in range(CPH):
            pltpu.sync_copy(rows_v.at[j], acc_sh.at[idx_v.at[g * CPH + j]], add=True)
    plsc.subcore_barrier()
    pltpu.sync_copy(acc_sh.at[pl.ds(s * NPT, NPT)], out_hbm.at[c].at[pl.ds(s * NPT, NPT)])


def _sc_scatter(msg4, idx3, zeros_nf):
    return pl.kernel(
        _scatter_body,
        out_type=jax.ShapeDtypeStruct((NC, N, F), jnp.float32),
        mesh=_sc_mesh(),
        scratch_types=[
            pltpu.VMEM((NCH, CHUNK), jnp.int32),
            pltpu.VMEM((CPH, CHUNK, F), jnp.float32),
            pltpu.VMEM_SHARED((N, F), jnp.float32),
            pltpu.SemaphoreType.DMA,
        ],
        compiler_params=pltpu.CompilerParams(use_tc_tiling_on_sc=False),
    )(msg4, idx3, zeros_nf)


def _msg_body(xj_ref, ea_ref, w_ref, out_ref):
    p = jnp.dot(xj_ref[...], w_ref[...], preferred_element_type=jnp.float32)
    ea = ea_ref[...]
    acc = p[:, ED * F:]
    for d in range(ED):
        acc = acc + ea[:, d:d + 1] * p[:, d * F:(d + 1) * F]
    out_ref[...] = acc


def _tc_msg(xj, ea, wbig):
    return pl.pallas_call(
        _msg_body,
        grid=(E // TB,),
        in_specs=[
            pl.BlockSpec((TB, F), lambda i: (i, 0)),
            pl.BlockSpec((TB, ED), lambda i: (i, 0)),
            pl.BlockSpec((F, (ED + 1) * F), lambda i: (0, 0)),
        ],
        out_specs=pl.BlockSpec((TB, F), lambda i: (i, 0)),
        out_shape=jax.ShapeDtypeStruct((E, F), jnp.float32),
    )(xj, ea, wbig)


def _combine_body(acc_ref, x_ref, root_ref, bias_ref, out_ref):
    r = jnp.dot(x_ref[...], root_ref[...], preferred_element_type=jnp.float32)
    out_ref[...] = jnp.maximum(acc_ref[0] + acc_ref[1] + r + bias_ref[...], 0.0)


def _tc_combine(acc, x, root, bias_row):
    return pl.pallas_call(
        _combine_body,
        out_shape=jax.ShapeDtypeStruct((N, F), jnp.float32),
    )(acc, x, root, bias_row)


def _final_body(acc_ref, h_ref, root_ref, bias_ref, batch_ref, wp1_ref, bp1_ref,
                wp2_ref, bp2_ref, out_ref):
    r = jnp.dot(h_ref[...], root_ref[...], preferred_element_type=jnp.float32)
    h2 = jnp.maximum(acc_ref[0] + acc_ref[1] + r + bias_ref[...], 0.0)
    seg = lax.broadcasted_iota(jnp.int32, (N, G), 1)
    mask = (batch_ref[...] == seg).astype(jnp.float32)
    sums = lax.dot_general(mask, h2, (((0,), (0,)), ((), ())),
                           preferred_element_type=jnp.float32)
    ones = jnp.ones((N, 1), jnp.float32)
    cnt = lax.dot_general(mask, ones, (((0,), (0,)), ((), ())),
                          preferred_element_type=jnp.float32)
    pooled = sums / jnp.maximum(cnt, 1.0)
    o = jnp.dot(pooled, wp1_ref[...], preferred_element_type=jnp.float32) + bp1_ref[...]
    o = jnp.dot(o, wp2_ref[...], preferred_element_type=jnp.float32) + bp2_ref[...]
    out_ref[...] = o


def _tc_final(acc, h1, root, bias_row, batch_col, wp1, bp1, wp2, bp2):
    return pl.pallas_call(
        _final_body,
        out_shape=jax.ShapeDtypeStruct((G, F), jnp.float32),
    )(acc, h1, root, bias_row, batch_col, wp1, bp1, wp2, bp2)


def _wbig(we, be):
    w = we.reshape(ED, F, F).transpose(1, 0, 2).reshape(F, ED * F)
    return jnp.concatenate([w, be.reshape(F, F)], axis=1)


def kernel(x, edge_index, edge_attr, batch, W1e, b1e, root1, bias1, W2e, b2e,
           root2, bias2, Wp1, bp1, Wp2, bp2):
    src3 = edge_index[0].reshape(NW, NCH, CHUNK)
    dst3 = edge_index[1].reshape(NW, NCH, CHUNK)
    ea = edge_attr.reshape(E, ED)
    zeros_nf = jnp.zeros((N, F), jnp.float32)
    wbig1 = _wbig(W1e, b1e)
    wbig2 = _wbig(W2e, b2e)
    batch_col = batch.reshape(N, 1)

    xj = _sc_gather(x, src3).reshape(E, F)
    msg = _tc_msg(xj, ea, wbig1)
    acc = _sc_scatter(msg.reshape(NW, NCH, CHUNK, F), dst3, zeros_nf)
    h1 = _tc_combine(acc, x, root1, bias1.reshape(1, F))

    hj = _sc_gather(h1, src3).reshape(E, F)
    msg2 = _tc_msg(hj, ea, wbig2)
    acc2 = _sc_scatter(msg2.reshape(NW, NCH, CHUNK, F), dst3, zeros_nf)

    return _tc_final(acc2, h1, root2, bias2.reshape(1, F), batch_col,
                     Wp1, bp1.reshape(1, F), Wp2, bp2.reshape(1, F))


# speedup vs baseline: 3.9178x; 2.0390x over previous
import functools

import jax
import jax.numpy as jnp
from jax import lax
from jax.experimental import pallas as pl
from jax.experimental.pallas import tpu as pltpu
from jax.experimental.pallas import tpu_sc as plsc

N = 10000
E = 160000
F = 32
ED = 16
G = 64

NC = 2
NS = 16
NW = NC * NS
EPW = E // NW
CHUNK = 125
NCH = EPW // CHUNK
CPH = 20
NGRP = NCH // CPH
NPT = N // NS

TB = 2000

@functools.cache
def _sc_mesh():
    return plsc.VectorSubcoreMesh(core_axis_name="c", subcore_axis_name="s",
                                  num_cores=NC, num_subcores=NS)


def _gather_body(table_hbm, idx_hbm, out_hbm, idx_v, rows_v, sem):
    c = lax.axis_index("c")
    s = lax.axis_index("s")
    wid = s * NC + c
    pltpu.sync_copy(idx_hbm.at[wid], idx_v)
    for g in range(NGRP):
        cps = [
            pltpu.async_copy(table_hbm.at[idx_v.at[g * CPH + j]], rows_v.at[j], sem)
            for j in range(CPH)
        ]
        for cp in cps:
            cp.wait()
        pltpu.sync_copy(rows_v, out_hbm.at[wid].at[pl.ds(g * CPH, CPH)])


def _sc_gather(table, idx3):
    return pl.kernel(
        _gather_body,
        out_type=jax.ShapeDtypeStruct((NW, NCH, CHUNK, F), jnp.float32),
        mesh=_sc_mesh(),
        scratch_types=[
            pltpu.VMEM((NCH, CHUNK), jnp.int32),
            pltpu.VMEM((CPH, CHUNK, F), jnp.float32),
            pltpu.SemaphoreType.DMA,
        ],
        compiler_params=pltpu.CompilerParams(use_tc_tiling_on_sc=False),
    )(table, idx3)


def _scatter_body(msg_hbm, idx_hbm, z_hbm, out_hbm, idx_v, rows_v, acc_sh, sem):
    c = lax.axis_index("c")
    s = lax.axis_index("s")
    wid = s * NC + c
    pltpu.sync_copy(z_hbm.at[pl.ds(s * NPT, NPT)], acc_sh.at[pl.ds(s * NPT, NPT)])
    pltpu.sync_copy(idx_hbm.at[wid], idx_v)
    plsc.subcore_barrier()
    for g in range(NGRP):
        pltpu.sync_copy(msg_hbm.at[wid].at[pl.ds(g * CPH, CPH)], rows_v)
        for j in range(CPH):
            pltpu.sync_copy(rows_v.at[j], acc_sh.at[idx_v.at[g * CPH + j]], add=True)
    plsc.subcore_barrier()
    pltpu.sync_copy(acc_sh.at[pl.ds(s * NPT, NPT)], out_hbm.at[c].at[pl.ds(s * NPT, NPT)])


def _sc_scatter(msg4, idx3, zeros_nf):
    return pl.kernel(
        _scatter_body,
        out_type=jax.ShapeDtypeStruct((NC, N, F), jnp.float32),
        mesh=_sc_mesh(),
        scratch_types=[
            pltpu.VMEM((NCH, CHUNK), jnp.int32),
            pltpu.VMEM((CPH, CHUNK, F), jnp.float32),
            pltpu.VMEM_SHARED((N, F), jnp.float32),
            pltpu.SemaphoreType.DMA,
        ],
        compiler_params=pltpu.CompilerParams(use_tc_tiling_on_sc=False),
    )(msg4, idx3, zeros_nf)


def _msg_body(xj_ref, ea_ref, w_ref, a_ref, s_ref, out_ref):
    p = jnp.dot(xj_ref[...], w_ref[...], preferred_element_type=jnp.float32)
    er = jnp.dot(ea_ref[...], a_ref[...], preferred_element_type=jnp.float32)
    out_ref[...] = jnp.dot(er * p, s_ref[...], preferred_element_type=jnp.float32)


def _tc_msg(xj, ea17, wbig, amat, smat):
    k = (ED + 1) * F
    return pl.pallas_call(
        _msg_body,
        grid=(E // TB,),
        in_specs=[
            pl.BlockSpec((TB, F), lambda i: (i, 0)),
            pl.BlockSpec((TB, ED + 1), lambda i: (i, 0)),
            pl.BlockSpec((F, k), lambda i: (0, 0)),
            pl.BlockSpec((ED + 1, k), lambda i: (0, 0)),
            pl.BlockSpec((k, F), lambda i: (0, 0)),
        ],
        out_specs=pl.BlockSpec((TB, F), lambda i: (i, 0)),
        out_shape=jax.ShapeDtypeStruct((E, F), jnp.float32),
    )(xj, ea17, wbig, amat, smat)


def _combine_body(acc_ref, x_ref, root_ref, bias_ref, out_ref):
    r = jnp.dot(x_ref[...], root_ref[...], preferred_element_type=jnp.float32)
    out_ref[...] = jnp.maximum(acc_ref[0] + acc_ref[1] + r + bias_ref[...], 0.0)


def _tc_combine(acc, x, root, bias_row):
    return pl.pallas_call(
        _combine_body,
        out_shape=jax.ShapeDtypeStruct((N, F), jnp.float32),
    )(acc, x, root, bias_row)


def _final_body(acc_ref, h_ref, root_ref, bias_ref, batch_ref, wp1_ref, bp1_ref,
                wp2_ref, bp2_ref, out_ref):
    r = jnp.dot(h_ref[...], root_ref[...], preferred_element_type=jnp.float32)
    h2 = jnp.maximum(acc_ref[0] + acc_ref[1] + r + bias_ref[...], 0.0)
    seg = lax.broadcasted_iota(jnp.int32, (N, G), 1)
    mask = (batch_ref[...] == seg).astype(jnp.float32)
    sums = lax.dot_general(mask, h2, (((0,), (0,)), ((), ())),
                           preferred_element_type=jnp.float32)
    ones = jnp.ones((N, 1), jnp.float32)
    cnt = lax.dot_general(mask, ones, (((0,), (0,)), ((), ())),
                          preferred_element_type=jnp.float32)
    pooled = sums / jnp.maximum(cnt, 1.0)
    o = jnp.dot(pooled, wp1_ref[...], preferred_element_type=jnp.float32) + bp1_ref[...]
    o = jnp.dot(o, wp2_ref[...], preferred_element_type=jnp.float32) + bp2_ref[...]
    out_ref[...] = o


def _tc_final(acc, h1, root, bias_row, batch_col, wp1, bp1, wp2, bp2):
    return pl.pallas_call(
        _final_body,
        out_shape=jax.ShapeDtypeStruct((G, F), jnp.float32),
    )(acc, h1, root, bias_row, batch_col, wp1, bp1, wp2, bp2)


def _wbig(we, be):
    w = we.reshape(ED, F, F).transpose(1, 0, 2).reshape(F, ED * F)
    return jnp.concatenate([w, be.reshape(F, F)], axis=1)


def kernel(x, edge_index, edge_attr, batch, W1e, b1e, root1, bias1, W2e, b2e,
           root2, bias2, Wp1, bp1, Wp2, bp2):
    src3 = edge_index[0].reshape(NW, NCH, CHUNK)
    dst3 = edge_index[1].reshape(NW, NCH, CHUNK)
    ea17 = jnp.concatenate([edge_attr, jnp.ones((E, 1), jnp.float32)], axis=1)
    zeros_nf = jnp.zeros((N, F), jnp.float32)
    wbig1 = _wbig(W1e, b1e)
    wbig2 = _wbig(W2e, b2e)
    amat = jnp.repeat(jnp.eye(ED + 1, dtype=jnp.float32), F, axis=1)
    smat = jnp.tile(jnp.eye(F, dtype=jnp.float32), (ED + 1, 1))
    batch_col = batch.reshape(N, 1)

    xj = _sc_gather(x, src3).reshape(E, F)
    msg = _tc_msg(xj, ea17, wbig1, amat, smat)
    acc = _sc_scatter(msg.reshape(NW, NCH, CHUNK, F), dst3, zeros_nf)
    h1 = _tc_combine(acc, x, root1, bias1.reshape(1, F))

    hj = _sc_gather(h1, src3).reshape(E, F)
    msg2 = _tc_msg(hj, ea17, wbig2, amat, smat)
    acc2 = _sc_scatter(msg2.reshape(NW, NCH, CHUNK, F), dst3, zeros_nf)

    return _tc_final(acc2, h1, root2, bias2.reshape(1, F), batch_col,
                     Wp1, bp1.reshape(1, F), Wp2, bp2.reshape(1, F))
